# baseline (speedup 1.0000x reference)
import functools

import jax
import jax.numpy as jnp
from jax import lax
from jax.experimental import pallas as pl
from jax.experimental.pallas import tpu as pltpu
from jax.experimental.pallas import tpu_sc as plsc

BATCH = 4096
SLOT = 26
NNZ = 20
VOCAB = 1000000
DIM = 32

NC = 2
NS = 16
NW = NC * NS
LANES = 16

SEGS = BATCH * SLOT
SPW = SEGS // NW
CSEG = 64
CKEY = CSEG * NNZ
NCHUNK = SPW // CSEG
GCHUNK = 128
NGATHER = CKEY // GCHUNK


def _emb_kernel(table_hbm, keys_hbm, out_hbm,
                idx0, idx1, rows0, rows1, out0, out1, sem0, sem1):
    wid = lax.axis_index("s") * NC + lax.axis_index("c")
    seg_base = wid * SPW
    key_base = seg_base * NNZ
    idx_v = (idx0, idx1)
    rows_v = (rows0, rows1)
    out_v = (out0, out1)
    sems = (sem0, sem1)

    def fire(chunk, b):
        pltpu.sync_copy(keys_hbm.at[pl.ds(key_base + chunk * CKEY, CKEY)],
                        idx_v[b])
        for j in range(NGATHER):
            pltpu.async_copy(
                table_hbm.at[idx_v[b].at[pl.ds(j * GCHUNK, GCHUNK)]],
                rows_v[b].at[pl.ds(j * GCHUNK, GCHUNK)],
                sems[b])

    def drain(b):
        for j in range(NGATHER):
            pltpu.make_async_copy(
                table_hbm.at[idx_v[b].at[pl.ds(j * GCHUNK, GCHUNK)]],
                rows_v[b].at[pl.ds(j * GCHUNK, GCHUNK)],
                sems[b]).wait()

    def reduce_store(chunk, b):
        rows = rows_v[b]
        out = out_v[b]

        @pl.loop(0, CSEG)
        def seg_body(s):
            r = s * NNZ
            acc0 = rows[r, pl.ds(0, LANES)]
            acc1 = rows[r, pl.ds(LANES, LANES)]
            for j in range(1, NNZ):
                acc0 = acc0 + rows[r + j, pl.ds(0, LANES)]
                acc1 = acc1 + rows[r + j, pl.ds(LANES, LANES)]
            out[s, pl.ds(0, LANES)] = acc0
            out[s, pl.ds(LANES, LANES)] = acc1

        pltpu.sync_copy(out, out_hbm.at[pl.ds(seg_base + chunk * CSEG, CSEG)])

    fire(0, 0)

    @pl.loop(0, NCHUNK, step=2)
    def chunk_body(cc):
        for b in range(2):
            c = cc + b
            nxt = c + 1

            @pl.when(nxt < NCHUNK)
            def _():
                fire(nxt, 1 - b)

            drain(b)
            reduce_store(c, b)


TW = 512
TGRID = (VOCAB + TW - 1) // TW
VPAD = TGRID * TW


def _transpose_body(tt_ref, out_ref):
    x = tt_ref[...]
    iv = lax.broadcasted_iota(jnp.int32, (TW, 128), 0)
    ij = lax.broadcasted_iota(jnp.int32, (TW, 128), 1)
    for q in range(4):
        sel = (iv == 4 * ij + q).astype(jnp.float32)
        part = jnp.dot(x, sel, preferred_element_type=jnp.float32,
                       precision=lax.Precision.HIGHEST)
        out_ref[:, 32 * q:32 * q + 32] = jnp.transpose(part)


def _linearize(table):
    tt = table.T
    return pl.pallas_call(
        _transpose_body,
        grid=(TGRID,),
        in_specs=[pl.BlockSpec((DIM, TW), lambda i: (0, i))],
        out_specs=pl.BlockSpec((TW * DIM // 128, 128), lambda i: (i, 0)),
        out_shape=jax.ShapeDtypeStruct((VPAD * DIM // 128, 128),
                                       jnp.float32),
    )(tt)


@jax.jit
def _run(keys, tflat):
    table = tflat.reshape(VPAD, DIM)
    mesh = plsc.VectorSubcoreMesh(
        core_axis_name="c", subcore_axis_name="s",
        num_cores=NC, num_subcores=NS)
    f = pl.kernel(
        _emb_kernel,
        out_type=jax.ShapeDtypeStruct((SEGS, DIM), jnp.float32),
        mesh=mesh,
        scratch_types=[
            pltpu.VMEM((CKEY,), jnp.int32),
            pltpu.VMEM((CKEY,), jnp.int32),
            pltpu.VMEM((CKEY, DIM), jnp.float32),
            pltpu.VMEM((CKEY, DIM), jnp.float32),
            pltpu.VMEM((CSEG, DIM), jnp.float32),
            pltpu.VMEM((CSEG, DIM), jnp.float32),
            pltpu.SemaphoreType.DMA,
            pltpu.SemaphoreType.DMA,
        ],
        compiler_params=pltpu.CompilerParams(use_tc_tiling_on_sc=False),
    )
    return f(table, keys)


def kernel(inputs, table):
    keys = inputs.reshape(-1)
    t128 = _linearize(table)
    out = _run(keys, t128.reshape(-1))
    return out.reshape(BATCH, SLOT, DIM)


# speedup vs baseline: 1.8418x; 1.8418x over previous
import functools

import jax
import jax.numpy as jnp
from jax import lax
from jax.experimental import pallas as pl
from jax.experimental.pallas import tpu as pltpu
from jax.experimental.pallas import tpu_sc as plsc

BATCH = 4096
SLOT = 26
NNZ = 20
VOCAB = 1000000
DIM = 32

NC = 2
NS = 16
NW = NC * NS
LANES = 16

SEGS = BATCH * SLOT
SPW = SEGS // NW
CSEG = 64
CKEY = CSEG * NNZ
NCHUNK = SPW // CSEG
GCHUNK = 128
NGATHER = CKEY // GCHUNK


def _emb_kernel(table_hbm, keys_hbm, out_hbm,
                idx0, idx1, rows0, rows1, out0, out1, sem0, sem1):
    wid = lax.axis_index("s") * NC + lax.axis_index("c")
    seg_base = wid * SPW
    key_base = seg_base * NNZ
    idx_v = (idx0, idx1)
    rows_v = (rows0, rows1)
    out_v = (out0, out1)
    sems = (sem0, sem1)

    def fire(chunk, b):
        pltpu.sync_copy(keys_hbm.at[pl.ds(key_base + chunk * CKEY, CKEY)],
                        idx_v[b])
        for j in range(NGATHER):
            pltpu.async_copy(
                table_hbm.at[idx_v[b].at[pl.ds(j * GCHUNK, GCHUNK)]],
                rows_v[b].at[pl.ds(j * GCHUNK, GCHUNK)],
                sems[b])

    def drain(b):
        for j in range(NGATHER):
            pltpu.make_async_copy(
                table_hbm.at[idx_v[b].at[pl.ds(j * GCHUNK, GCHUNK)]],
                rows_v[b].at[pl.ds(j * GCHUNK, GCHUNK)],
                sems[b]).wait()

    def reduce_store(chunk, b):
        rows = rows_v[b]
        out = out_v[b]

        @pl.loop(0, CSEG)
        def seg_body(s):
            r = s * NNZ
            acc0 = rows[r, pl.ds(0, LANES)]
            acc1 = rows[r, pl.ds(LANES, LANES)]
            for j in range(1, NNZ):
                acc0 = acc0 + rows[r + j, pl.ds(0, LANES)]
                acc1 = acc1 + rows[r + j, pl.ds(LANES, LANES)]
            out[s, pl.ds(0, LANES)] = acc0
            out[s, pl.ds(LANES, LANES)] = acc1

        pltpu.sync_copy(out, out_hbm.at[pl.ds(seg_base + chunk * CSEG, CSEG)])

    fire(0, 0)

    @pl.loop(0, NCHUNK, step=2)
    def chunk_body(cc):
        for b in range(2):
            c = cc + b
            nxt = c + 1

            @pl.when(nxt < NCHUNK)
            def _():
                fire(nxt, 1 - b)

            drain(b)
            reduce_store(c, b)


TB = 128
NBF = VOCAB // TB
TAIL = VOCAB - NBF * TB
BPW = NBF // NW
BXTRA = NBF - BPW * NW


def _tr_kernel(tt_hbm, out_hbm, in0, in1, int_, ov, sem0, sem1):
    wid = lax.axis_index("s") * NC + lax.axis_index("c")
    extra = wid < BXTRA
    start = wid * BPW + jnp.minimum(wid, BXTRA)
    iota = lax.iota(jnp.int32, 16)
    bufs = (in0, in1)
    sems = (sem0, sem1)

    def fire(b, u):
        v0 = (start + b) * TB
        pltpu.async_copy(tt_hbm.at[pl.ds(0, DIM), pl.ds(v0, TB)],
                         bufs[u], sems[u])

    def drain(u):
        pltpu.make_async_copy(tt_hbm.at[pl.ds(0, DIM), pl.ds(0, TB)],
                              bufs[u], sems[u]).wait()

    def extract_store(b, u, width):
        src = bufs[u]

        @pl.loop(0, width)
        def col(l):
            cv = jnp.zeros((16,), jnp.int32) + l
            ov[pl.ds(l * DIM, 16)] = plsc.load_gather(src, [iota, cv])
            ov[pl.ds(l * DIM + 16, 16)] = plsc.load_gather(
                src, [iota + 16, cv])

        v0 = (start + b) * TB
        pltpu.sync_copy(ov.at[pl.ds(0, width * DIM)],
                        out_hbm.at[pl.ds(v0 * DIM, width * DIM)])

    fire(0, 0)

    @pl.loop(0, BPW, step=2)
    def pair(b):
        fire(b + 1, 1)
        drain(0)
        extract_store(b, 0, TB)

        @pl.when(jnp.logical_or(b + 2 < BPW, extra))
        def _():
            fire(b + 2, 0)

        drain(1)
        extract_store(b + 1, 1, TB)

    @pl.when(extra)
    def _():
        drain(0)
        extract_store(BPW, 0, TB)

    @pl.when(wid == NW - 1)
    def _():
        pltpu.sync_copy(tt_hbm.at[pl.ds(0, DIM), pl.ds(NBF * TB, TAIL)],
                        int_)

        @pl.loop(0, TAIL)
        def tcol(l):
            cv = jnp.zeros((16,), jnp.int32) + l
            ov[pl.ds(l * DIM, 16)] = plsc.load_gather(int_, [iota, cv])
            ov[pl.ds(l * DIM + 16, 16)] = plsc.load_gather(
                int_, [iota + 16, cv])

        pltpu.sync_copy(ov.at[pl.ds(0, TAIL * DIM)],
                        out_hbm.at[pl.ds(NBF * TB * DIM, TAIL * DIM)])


def _sc_transpose(table):
    mesh = plsc.VectorSubcoreMesh(
        core_axis_name="c", subcore_axis_name="s",
        num_cores=NC, num_subcores=NS)
    f = pl.kernel(
        _tr_kernel,
        out_type=jax.ShapeDtypeStruct((VOCAB * DIM,), jnp.float32),
        mesh=mesh,
        scratch_types=[
            pltpu.VMEM((DIM, TB), jnp.float32),
            pltpu.VMEM((DIM, TB), jnp.float32),
            pltpu.VMEM((DIM, TAIL), jnp.float32),
            pltpu.VMEM((TB * DIM,), jnp.float32),
            pltpu.SemaphoreType.DMA,
            pltpu.SemaphoreType.DMA,
        ],
        compiler_params=pltpu.CompilerParams(needs_layout_passes=False),
    )
    return f(table.T)


@jax.jit
def _run(keys, tflat):
    table = tflat.reshape(VOCAB, DIM)
    mesh = plsc.VectorSubcoreMesh(
        core_axis_name="c", subcore_axis_name="s",
        num_cores=NC, num_subcores=NS)
    f = pl.kernel(
        _emb_kernel,
        out_type=jax.ShapeDtypeStruct((SEGS, DIM), jnp.float32),
        mesh=mesh,
        scratch_types=[
            pltpu.VMEM((CKEY,), jnp.int32),
            pltpu.VMEM((CKEY,), jnp.int32),
            pltpu.VMEM((CKEY, DIM), jnp.float32),
            pltpu.VMEM((CKEY, DIM), jnp.float32),
            pltpu.VMEM((CSEG, DIM), jnp.float32),
            pltpu.VMEM((CSEG, DIM), jnp.float32),
            pltpu.SemaphoreType.DMA,
            pltpu.SemaphoreType.DMA,
        ],
        compiler_params=pltpu.CompilerParams(use_tc_tiling_on_sc=False),
    )
    return f(table, keys)


def kernel(inputs, table):
    keys = inputs.reshape(-1)
    out = _run(keys, _sc_transpose(table))
    return out.reshape(BATCH, SLOT, DIM)


# speedup vs baseline: 2.1302x; 1.1566x over previous
import functools

import jax
import jax.numpy as jnp
from jax import lax
from jax.experimental import pallas as pl
from jax.experimental.pallas import tpu as pltpu
from jax.experimental.pallas import tpu_sc as plsc

BATCH = 4096
SLOT = 26
NNZ = 20
VOCAB = 1000000
DIM = 32

NC = 2
NS = 16
NW = NC * NS
LANES = 16

SEGS = BATCH * SLOT
SPW = SEGS // NW
CSEG = 64
CKEY = CSEG * NNZ
NCHUNK = SPW // CSEG
GCHUNK = 128
NGATHER = CKEY // GCHUNK


def _emb_kernel(table_hbm, keys_hbm, out_hbm,
                idx0, idx1, rows0, rows1, out0, out1, sem0, sem1):
    wid = lax.axis_index("s") * NC + lax.axis_index("c")
    seg_base = wid * SPW
    key_base = seg_base * NNZ
    idx_v = (idx0, idx1)
    rows_v = (rows0, rows1)
    out_v = (out0, out1)
    sems = (sem0, sem1)

    def fire(chunk, b):
        pltpu.sync_copy(keys_hbm.at[pl.ds(key_base + chunk * CKEY, CKEY)],
                        idx_v[b])
        for j in range(NGATHER):
            pltpu.async_copy(
                table_hbm.at[idx_v[b].at[pl.ds(j * GCHUNK, GCHUNK)]],
                rows_v[b].at[pl.ds(j * GCHUNK, GCHUNK)],
                sems[b])

    def drain(b):
        for j in range(NGATHER):
            pltpu.make_async_copy(
                table_hbm.at[idx_v[b].at[pl.ds(j * GCHUNK, GCHUNK)]],
                rows_v[b].at[pl.ds(j * GCHUNK, GCHUNK)],
                sems[b]).wait()

    def reduce_store(chunk, b):
        rows = rows_v[b]
        out = out_v[b]

        @pl.loop(0, CSEG)
        def seg_body(s):
            r = s * NNZ
            acc0 = rows[r, pl.ds(0, LANES)]
            acc1 = rows[r, pl.ds(LANES, LANES)]
            for j in range(1, NNZ):
                acc0 = acc0 + rows[r + j, pl.ds(0, LANES)]
                acc1 = acc1 + rows[r + j, pl.ds(LANES, LANES)]
            out[s, pl.ds(0, LANES)] = acc0
            out[s, pl.ds(LANES, LANES)] = acc1

        pltpu.sync_copy(out, out_hbm.at[pl.ds(seg_base + chunk * CSEG, CSEG)])

    fire(0, 0)

    @pl.loop(0, NCHUNK, step=2)
    def chunk_body(cc):
        for b in range(2):
            c = cc + b
            nxt = c + 1

            @pl.when(nxt < NCHUNK)
            def _():
                fire(nxt, 1 - b)

            drain(b)
            reduce_store(c, b)


TB = 128
NBF = VOCAB // TB
TAIL = VOCAB - NBF * TB
BPW = NBF // NW
BXTRA = NBF - BPW * NW


def _tr_kernel(tt_hbm, out_hbm, in0, in1, int_, ov, sem0, sem1):
    wid = lax.axis_index("s") * NC + lax.axis_index("c")
    extra = wid < BXTRA
    start = wid * BPW + jnp.minimum(wid, BXTRA)
    iota = lax.iota(jnp.int32, 16)
    bufs = (in0, in1)
    sems = (sem0, sem1)

    def fire(b, u):
        v0 = (start + b) * TB
        pltpu.async_copy(tt_hbm.at[pl.ds(0, DIM), pl.ds(v0, TB)],
                         bufs[u], sems[u])

    def drain(u):
        pltpu.make_async_copy(tt_hbm.at[pl.ds(0, DIM), pl.ds(0, TB)],
                              bufs[u], sems[u]).wait()

    iota32 = iota * DIM

    def extract(src, width):
        @pl.loop(0, width, step=16)
        def colg(l0):
            base = iota32 + l0 * DIM
            for d in range(DIM):
                row = src[d, pl.ds(l0, 16)]
                plsc.store_scatter(ov, [base + d], row)

    def extract_store(b, u, width):
        extract(bufs[u], width)
        v0 = (start + b) * TB
        pltpu.sync_copy(ov.at[pl.ds(0, width * DIM)],
                        out_hbm.at[pl.ds(v0 * DIM, width * DIM)])

    fire(0, 0)

    @pl.loop(0, BPW, step=2)
    def pair(b):
        fire(b + 1, 1)
        drain(0)
        extract_store(b, 0, TB)

        @pl.when(jnp.logical_or(b + 2 < BPW, extra))
        def _():
            fire(b + 2, 0)

        drain(1)
        extract_store(b + 1, 1, TB)

    @pl.when(extra)
    def _():
        drain(0)
        extract_store(BPW, 0, TB)

    @pl.when(wid == NW - 1)
    def _():
        pltpu.sync_copy(tt_hbm.at[pl.ds(0, DIM), pl.ds(NBF * TB, TAIL)],
                        int_)
        extract(int_, TAIL)
        pltpu.sync_copy(ov.at[pl.ds(0, TAIL * DIM)],
                        out_hbm.at[pl.ds(NBF * TB * DIM, TAIL * DIM)])


def _sc_transpose(table):
    mesh = plsc.VectorSubcoreMesh(
        core_axis_name="c", subcore_axis_name="s",
        num_cores=NC, num_subcores=NS)
    f = pl.kernel(
        _tr_kernel,
        out_type=jax.ShapeDtypeStruct((VOCAB * DIM,), jnp.float32),
        mesh=mesh,
        scratch_types=[
            pltpu.VMEM((DIM, TB), jnp.float32),
            pltpu.VMEM((DIM, TB), jnp.float32),
            pltpu.VMEM((DIM, TAIL), jnp.float32),
            pltpu.VMEM((TB * DIM,), jnp.float32),
            pltpu.SemaphoreType.DMA,
            pltpu.SemaphoreType.DMA,
        ],
        compiler_params=pltpu.CompilerParams(needs_layout_passes=False),
    )
    return f(table.T)


@jax.jit
def _run(keys, tflat):
    table = tflat.reshape(VOCAB, DIM)
    mesh = plsc.VectorSubcoreMesh(
        core_axis_name="c", subcore_axis_name="s",
        num_cores=NC, num_subcores=NS)
    f = pl.kernel(
        _emb_kernel,
        out_type=jax.ShapeDtypeStruct((SEGS, DIM), jnp.float32),
        mesh=mesh,
        scratch_types=[
            pltpu.VMEM((CKEY,), jnp.int32),
            pltpu.VMEM((CKEY,), jnp.int32),
            pltpu.VMEM((CKEY, DIM), jnp.float32),
            pltpu.VMEM((CKEY, DIM), jnp.float32),
            pltpu.VMEM((CSEG, DIM), jnp.float32),
            pltpu.VMEM((CSEG, DIM), jnp.float32),
            pltpu.SemaphoreType.DMA,
            pltpu.SemaphoreType.DMA,
        ],
        compiler_params=pltpu.CompilerParams(use_tc_tiling_on_sc=False),
    )
    return f(table, keys)


def kernel(inputs, table):
    keys = inputs.reshape(-1)
    out = _run(keys, _sc_transpose(table))
    return out.reshape(BATCH, SLOT, DIM)


# speedup vs baseline: 2.2516x; 1.0570x over previous
import functools

import jax
import jax.numpy as jnp
from jax import lax
from jax.experimental import pallas as pl
from jax.experimental.pallas import tpu as pltpu
from jax.experimental.pallas import tpu_sc as plsc

BATCH = 4096
SLOT = 26
NNZ = 20
VOCAB = 1000000
DIM = 32

NC = 2
NS = 16
NW = NC * NS
LANES = 16

SEGS = BATCH * SLOT
SPW = SEGS // NW
CSEG = 64
CKEY = CSEG * NNZ
NCHUNK = SPW // CSEG
GCHUNK = 128
NGATHER = CKEY // GCHUNK


def _emb_kernel(table_hbm, keys_hbm, out_hbm,
                idx0, idx1, rows0, rows1, out0, out1, sem0, sem1):
    wid = lax.axis_index("s") * NC + lax.axis_index("c")
    seg_base = wid * SPW
    key_base = seg_base * NNZ
    idx_v = (idx0, idx1)
    rows_v = (rows0, rows1)
    out_v = (out0, out1)
    sems = (sem0, sem1)

    def fire(chunk, b):
        pltpu.sync_copy(keys_hbm.at[pl.ds(key_base + chunk * CKEY, CKEY)],
                        idx_v[b])
        for j in range(NGATHER):
            pltpu.async_copy(
                table_hbm.at[idx_v[b].at[pl.ds(j * GCHUNK, GCHUNK)]],
                rows_v[b].at[pl.ds(j * GCHUNK, GCHUNK)],
                sems[b])

    def drain(b):
        for j in range(NGATHER):
            pltpu.make_async_copy(
                table_hbm.at[idx_v[b].at[pl.ds(j * GCHUNK, GCHUNK)]],
                rows_v[b].at[pl.ds(j * GCHUNK, GCHUNK)],
                sems[b]).wait()

    def reduce_store(chunk, b):
        rows = rows_v[b]
        out = out_v[b]

        @pl.loop(0, CSEG)
        def seg_body(s):
            r = s * NNZ
            acc0 = rows[r, pl.ds(0, LANES)]
            acc1 = rows[r, pl.ds(LANES, LANES)]
            for j in range(1, NNZ):
                acc0 = acc0 + rows[r + j, pl.ds(0, LANES)]
                acc1 = acc1 + rows[r + j, pl.ds(LANES, LANES)]
            out[s, pl.ds(0, LANES)] = acc0
            out[s, pl.ds(LANES, LANES)] = acc1

        pltpu.sync_copy(out, out_hbm.at[pl.ds(seg_base + chunk * CSEG, CSEG)])

    fire(0, 0)

    @pl.loop(0, NCHUNK, step=2)
    def chunk_body(cc):
        for b in range(2):
            c = cc + b
            nxt = c + 1

            @pl.when(nxt < NCHUNK)
            def _():
                fire(nxt, 1 - b)

            drain(b)
            reduce_store(c, b)


TB = 256
NBF = VOCAB // TB
TAIL = VOCAB - NBF * TB
BPW = NBF // NW
BXTRA = NBF - BPW * NW


def _tr_kernel(tt_hbm, out_hbm, in0, in1, int_, ov0, ov1,
               sem0, sem1, semo0, semo1):
    wid = lax.axis_index("s") * NC + lax.axis_index("c")
    extra = wid < BXTRA
    tailw = wid == NW - 1
    start = wid * BPW + jnp.minimum(wid, BXTRA)
    iota = lax.iota(jnp.int32, 16)
    bufs = (in0, in1)
    sems = (sem0, sem1)
    ovs = (ov0, ov1)
    semo = (semo0, semo1)

    def fire(b, u):
        v0 = (start + b) * TB
        pltpu.async_copy(tt_hbm.at[pl.ds(0, DIM), pl.ds(v0, TB)],
                         bufs[u], sems[u])

    def drain(u):
        pltpu.make_async_copy(tt_hbm.at[pl.ds(0, DIM), pl.ds(0, TB)],
                              bufs[u], sems[u]).wait()

    def fire_out(b, u, width):
        v0 = (start + b) * TB
        pltpu.async_copy(ovs[u].at[pl.ds(0, width * DIM)],
                         out_hbm.at[pl.ds(v0 * DIM, width * DIM)],
                         semo[u])

    def drain_out(u, width):
        pltpu.make_async_copy(ovs[u].at[pl.ds(0, width * DIM)],
                              out_hbm.at[pl.ds(0, width * DIM)],
                              semo[u]).wait()

    iota32 = iota * DIM

    def extract(src, ov, width):
        @pl.loop(0, width, step=16)
        def colg(l0):
            base = iota32 + l0 * DIM
            for d in range(DIM):
                row = src[d, pl.ds(l0, 16)]
                plsc.store_scatter(ov, [base + d], row)

    fire(0, 0)

    @pl.loop(0, BPW, step=2)
    def pair(b):
        fire(b + 1, 1)
        drain(0)

        @pl.when(b >= 2)
        def _():
            drain_out(0, TB)

        extract(in0, ov0, TB)
        fire_out(b, 0, TB)

        @pl.when(jnp.logical_or(b + 2 < BPW, extra))
        def _():
            fire(b + 2, 0)

        drain(1)

        @pl.when(b >= 2)
        def _():
            drain_out(1, TB)

        extract(in1, ov1, TB)
        fire_out(b + 1, 1, TB)

    @pl.when(extra)
    def _():
        drain(0)
        drain_out(0, TB)
        extract(in0, ov0, TB)
        fire_out(BPW, 0, TB)

    @pl.when(tailw)
    def _():
        pltpu.sync_copy(tt_hbm.at[pl.ds(0, DIM), pl.ds(NBF * TB, TAIL)],
                        int_)
        drain_out(1, TB)
        extract(int_, ov1, TAIL)
        pltpu.async_copy(ov1.at[pl.ds(0, TAIL * DIM)],
                         out_hbm.at[pl.ds(NBF * TB * DIM, TAIL * DIM)],
                         semo1)

    drain_out(0, TB)

    @pl.when(tailw)
    def _():
        drain_out(1, TAIL)

    @pl.when(jnp.logical_not(tailw))
    def _():
        drain_out(1, TB)


def _sc_transpose(table):
    mesh = plsc.VectorSubcoreMesh(
        core_axis_name="c", subcore_axis_name="s",
        num_cores=NC, num_subcores=NS)
    f = pl.kernel(
        _tr_kernel,
        out_type=jax.ShapeDtypeStruct((VOCAB * DIM,), jnp.float32),
        mesh=mesh,
        scratch_types=[
            pltpu.VMEM((DIM, TB), jnp.float32),
            pltpu.VMEM((DIM, TB), jnp.float32),
            pltpu.VMEM((DIM, TAIL), jnp.float32),
            pltpu.VMEM((TB * DIM,), jnp.float32),
            pltpu.VMEM((TB * DIM,), jnp.float32),
            pltpu.SemaphoreType.DMA,
            pltpu.SemaphoreType.DMA,
            pltpu.SemaphoreType.DMA,
            pltpu.SemaphoreType.DMA,
        ],
        compiler_params=pltpu.CompilerParams(needs_layout_passes=False),
    )
    return f(table.T)


@jax.jit
def _run(keys, tflat):
    table = tflat.reshape(VOCAB, DIM)
    mesh = plsc.VectorSubcoreMesh(
        core_axis_name="c", subcore_axis_name="s",
        num_cores=NC, num_subcores=NS)
    f = pl.kernel(
        _emb_kernel,
        out_type=jax.ShapeDtypeStruct((SEGS, DIM), jnp.float32),
        mesh=mesh,
        scratch_types=[
            pltpu.VMEM((CKEY,), jnp.int32),
            pltpu.VMEM((CKEY,), jnp.int32),
            pltpu.VMEM((CKEY, DIM), jnp.float32),
            pltpu.VMEM((CKEY, DIM), jnp.float32),
            pltpu.VMEM((CSEG, DIM), jnp.float32),
            pltpu.VMEM((CSEG, DIM), jnp.float32),
            pltpu.SemaphoreType.DMA,
            pltpu.SemaphoreType.DMA,
        ],
        compiler_params=pltpu.CompilerParams(use_tc_tiling_on_sc=False),
    )
    return f(table, keys)


def kernel(inputs, table):
    keys = inputs.reshape(-1)
    out = _run(keys, _sc_transpose(table))
    return out.reshape(BATCH, SLOT, DIM)


# speedup vs baseline: 2.7975x; 1.2425x over previous
import functools

import jax
import jax.numpy as jnp
from jax import lax
from jax.experimental import pallas as pl
from jax.experimental.pallas import tpu as pltpu
from jax.experimental.pallas import tpu_sc as plsc

BATCH = 4096
SLOT = 26
NNZ = 20
VOCAB = 1000000
DIM = 32

NC = 2
NS = 16
NW = NC * NS
LANES = 16

SEGS = BATCH * SLOT
SPW = SEGS // NW
CSEG = 64
CKEY = CSEG * NNZ
NCHUNK = SPW // CSEG
GCHUNK = 128
NGATHER = CKEY // GCHUNK


def _emb_kernel(table_hbm, keys_hbm, out_hbm,
                idx0, idx1, rows0, rows1, out0, out1, sem0, sem1):
    wid = lax.axis_index("s") * NC + lax.axis_index("c")
    seg_base = wid * SPW
    key_base = seg_base * NNZ
    idx_v = (idx0, idx1)
    rows_v = (rows0, rows1)
    out_v = (out0, out1)
    sems = (sem0, sem1)

    def fire(chunk, b):
        pltpu.sync_copy(keys_hbm.at[pl.ds(key_base + chunk * CKEY, CKEY)],
                        idx_v[b])
        for j in range(NGATHER):
            pltpu.async_copy(
                table_hbm.at[idx_v[b].at[pl.ds(j * GCHUNK, GCHUNK)]],
                rows_v[b].at[pl.ds(j * GCHUNK, GCHUNK)],
                sems[b])

    def drain(b):
        for j in range(NGATHER):
            pltpu.make_async_copy(
                table_hbm.at[idx_v[b].at[pl.ds(j * GCHUNK, GCHUNK)]],
                rows_v[b].at[pl.ds(j * GCHUNK, GCHUNK)],
                sems[b]).wait()

    def reduce_store(chunk, b):
        rows = rows_v[b]
        out = out_v[b]

        @pl.loop(0, CSEG)
        def seg_body(s):
            r = s * NNZ
            acc0 = rows[r, pl.ds(0, LANES)]
            acc1 = rows[r, pl.ds(LANES, LANES)]
            for j in range(1, NNZ):
                acc0 = acc0 + rows[r + j, pl.ds(0, LANES)]
                acc1 = acc1 + rows[r + j, pl.ds(LANES, LANES)]
            out[s, pl.ds(0, LANES)] = acc0
            out[s, pl.ds(LANES, LANES)] = acc1

        pltpu.sync_copy(out, out_hbm.at[pl.ds(seg_base + chunk * CSEG, CSEG)])

    fire(0, 0)

    @pl.loop(0, NCHUNK, step=2)
    def chunk_body(cc):
        for b in range(2):
            c = cc + b
            nxt = c + 1

            @pl.when(nxt < NCHUNK)
            def _():
                fire(nxt, 1 - b)

            drain(b)
            reduce_store(c, b)


TB = 256
NBF = VOCAB // TB
TAIL = VOCAB - NBF * TB
BPW = NBF // NW
BXTRA = NBF - BPW * NW


def _tr_kernel(tt_hbm, out_hbm, in0, in1, int_, ov33, ov0, ov1,
               sem0, sem1, semo0, semo1):
    wid = lax.axis_index("s") * NC + lax.axis_index("c")
    extra = wid < BXTRA
    tailw = wid == NW - 1
    start = wid * BPW + jnp.minimum(wid, BXTRA)
    iota = lax.iota(jnp.int32, 16)
    bufs = (in0, in1)
    sems = (sem0, sem1)
    ovs = (ov0, ov1)
    semo = (semo0, semo1)

    def fire(b, u):
        v0 = (start + b) * TB
        pltpu.async_copy(tt_hbm.at[pl.ds(0, DIM), pl.ds(v0, TB)],
                         bufs[u], sems[u])

    def drain(u):
        pltpu.make_async_copy(tt_hbm.at[pl.ds(0, DIM), pl.ds(0, TB)],
                              bufs[u], sems[u]).wait()

    def fire_out(b, u, width):
        v0 = (start + b) * TB
        pltpu.async_copy(ovs[u].at[pl.ds(0, width * DIM)],
                         out_hbm.at[pl.ds(v0 * DIM, width * DIM)],
                         semo[u])

    def drain_out(u, width):
        pltpu.make_async_copy(ovs[u].at[pl.ds(0, width * DIM)],
                              out_hbm.at[pl.ds(0, width * DIM)],
                              semo[u]).wait()

    iota33 = iota * (DIM + 1)

    def extract(src, ov, width):
        @pl.loop(0, width, step=16)
        def colg(l0):
            base = iota33 + l0 * (DIM + 1)
            for d in range(DIM):
                row = src[d, pl.ds(l0, 16)]
                plsc.store_scatter(ov33, [base + d], row)

        @pl.loop(0, width, unroll=4)
        def unpad(l):
            s = l * (DIM + 1)
            t = l * DIM
            ov[pl.ds(t, 16)] = ov33[pl.ds(s, 16)]
            ov[pl.ds(t + 16, 16)] = ov33[pl.ds(s + 16, 16)]

    fire(0, 0)

    @pl.loop(0, BPW, step=2)
    def pair(b):
        fire(b + 1, 1)
        drain(0)

        @pl.when(b >= 2)
        def _():
            drain_out(0, TB)

        extract(in0, ov0, TB)
        fire_out(b, 0, TB)

        @pl.when(jnp.logical_or(b + 2 < BPW, extra))
        def _():
            fire(b + 2, 0)

        drain(1)

        @pl.when(b >= 2)
        def _():
            drain_out(1, TB)

        extract(in1, ov1, TB)
        fire_out(b + 1, 1, TB)

    @pl.when(extra)
    def _():
        drain(0)
        drain_out(0, TB)
        extract(in0, ov0, TB)
        fire_out(BPW, 0, TB)

    @pl.when(tailw)
    def _():
        pltpu.sync_copy(tt_hbm.at[pl.ds(0, DIM), pl.ds(NBF * TB, TAIL)],
                        int_)
        drain_out(1, TB)
        extract(int_, ov1, TAIL)
        pltpu.async_copy(ov1.at[pl.ds(0, TAIL * DIM)],
                         out_hbm.at[pl.ds(NBF * TB * DIM, TAIL * DIM)],
                         semo1)

    drain_out(0, TB)

    @pl.when(tailw)
    def _():
        drain_out(1, TAIL)

    @pl.when(jnp.logical_not(tailw))
    def _():
        drain_out(1, TB)


def _sc_transpose(table):
    mesh = plsc.VectorSubcoreMesh(
        core_axis_name="c", subcore_axis_name="s",
        num_cores=NC, num_subcores=NS)
    f = pl.kernel(
        _tr_kernel,
        out_type=jax.ShapeDtypeStruct((VOCAB * DIM,), jnp.float32),
        mesh=mesh,
        scratch_types=[
            pltpu.VMEM((DIM, TB), jnp.float32),
            pltpu.VMEM((DIM, TB), jnp.float32),
            pltpu.VMEM((DIM, TAIL), jnp.float32),
            pltpu.VMEM((TB * (DIM + 1),), jnp.float32),
            pltpu.VMEM((TB * DIM,), jnp.float32),
            pltpu.VMEM((TB * DIM,), jnp.float32),
            pltpu.SemaphoreType.DMA,
            pltpu.SemaphoreType.DMA,
            pltpu.SemaphoreType.DMA,
            pltpu.SemaphoreType.DMA,
        ],
        compiler_params=pltpu.CompilerParams(needs_layout_passes=False),
    )
    return f(table.T)


@jax.jit
def _run(keys, tflat):
    table = tflat.reshape(VOCAB, DIM)
    mesh = plsc.VectorSubcoreMesh(
        core_axis_name="c", subcore_axis_name="s",
        num_cores=NC, num_subcores=NS)
    f = pl.kernel(
        _emb_kernel,
        out_type=jax.ShapeDtypeStruct((SEGS, DIM), jnp.float32),
        mesh=mesh,
        scratch_types=[
            pltpu.VMEM((CKEY,), jnp.int32),
            pltpu.VMEM((CKEY,), jnp.int32),
            pltpu.VMEM((CKEY, DIM), jnp.float32),
            pltpu.VMEM((CKEY, DIM), jnp.float32),
            pltpu.VMEM((CSEG, DIM), jnp.float32),
            pltpu.VMEM((CSEG, DIM), jnp.float32),
            pltpu.SemaphoreType.DMA,
            pltpu.SemaphoreType.DMA,
        ],
        compiler_params=pltpu.CompilerParams(use_tc_tiling_on_sc=False),
    )
    return f(table, keys)


def kernel(inputs, table):
    keys = inputs.reshape(-1)
    out = _run(keys, _sc_transpose(table))
    return out.reshape(BATCH, SLOT, DIM)


# speedup vs baseline: 2.8656x; 1.0244x over previous
import functools

import jax
import jax.numpy as jnp
from jax import lax
from jax.experimental import pallas as pl
from jax.experimental.pallas import tpu as pltpu
from jax.experimental.pallas import tpu_sc as plsc

BATCH = 4096
SLOT = 26
NNZ = 20
VOCAB = 1000000
DIM = 32

NC = 2
NS = 16
NW = NC * NS
LANES = 16

SEGS = BATCH * SLOT
SPW = SEGS // NW
CSEG = 64
CKEY = CSEG * NNZ
NCHUNK = SPW // CSEG
GCHUNK = 128
NGATHER = CKEY // GCHUNK


def _emb_kernel(table_hbm, keys_hbm, out_hbm,
                idx0, idx1, rows0, rows1, out0, out1, sem0, sem1):
    wid = lax.axis_index("s") * NC + lax.axis_index("c")
    seg_base = wid * SPW
    key_base = seg_base * NNZ
    idx_v = (idx0, idx1)
    rows_v = (rows0, rows1)
    out_v = (out0, out1)
    sems = (sem0, sem1)

    def fire(chunk, b):
        pltpu.sync_copy(keys_hbm.at[pl.ds(key_base + chunk * CKEY, CKEY)],
                        idx_v[b])
        for j in range(NGATHER):
            pltpu.async_copy(
                table_hbm.at[idx_v[b].at[pl.ds(j * GCHUNK, GCHUNK)]],
                rows_v[b].at[pl.ds(j * GCHUNK, GCHUNK)],
                sems[b])

    def drain(b):
        for j in range(NGATHER):
            pltpu.make_async_copy(
                table_hbm.at[idx_v[b].at[pl.ds(j * GCHUNK, GCHUNK)]],
                rows_v[b].at[pl.ds(j * GCHUNK, GCHUNK)],
                sems[b]).wait()

    def reduce_store(chunk, b):
        rows = rows_v[b]
        out = out_v[b]

        @pl.loop(0, CSEG)
        def seg_body(s):
            r = s * NNZ
            acc0 = rows[r, pl.ds(0, LANES)]
            acc1 = rows[r, pl.ds(LANES, LANES)]
            for j in range(1, NNZ):
                acc0 = acc0 + rows[r + j, pl.ds(0, LANES)]
                acc1 = acc1 + rows[r + j, pl.ds(LANES, LANES)]
            out[s, pl.ds(0, LANES)] = acc0
            out[s, pl.ds(LANES, LANES)] = acc1

        pltpu.sync_copy(out, out_hbm.at[pl.ds(seg_base + chunk * CSEG, CSEG)])

    fire(0, 0)

    @pl.loop(0, NCHUNK, step=2)
    def chunk_body(cc):
        for b in range(2):
            c = cc + b
            nxt = c + 1

            @pl.when(nxt < NCHUNK)
            def _():
                fire(nxt, 1 - b)

            drain(b)
            reduce_store(c, b)


TB = 256
NBF = VOCAB // TB
TAIL = VOCAB - NBF * TB
BPW = NBF // NW
BXTRA = NBF - BPW * NW


def _tr_kernel(tt_hbm, out_hbm, in0, in1, int_, ov33, ov0, ov1,
               sem0, sem1, semo0, semo1):
    wid = lax.axis_index("s") * NC + lax.axis_index("c")
    extra = wid < BXTRA
    tailw = wid == NW - 1
    start = wid * BPW + jnp.minimum(wid, BXTRA)
    iota = lax.iota(jnp.int32, 16)
    bufs = (in0, in1)
    sems = (sem0, sem1)
    ovs = (ov0, ov1)
    semo = (semo0, semo1)

    def fire(b, u):
        v0 = (start + b) * TB
        pltpu.async_copy(tt_hbm.at[pl.ds(0, DIM), pl.ds(v0, TB)],
                         bufs[u], sems[u])

    def drain(u):
        pltpu.make_async_copy(tt_hbm.at[pl.ds(0, DIM), pl.ds(0, TB)],
                              bufs[u], sems[u]).wait()

    def fire_out(b, u, width):
        v0 = (start + b) * TB
        pltpu.async_copy(ovs[u].at[pl.ds(0, width * DIM)],
                         out_hbm.at[pl.ds(v0 * DIM, width * DIM)],
                         semo[u])

    def drain_out(u, width):
        pltpu.make_async_copy(ovs[u].at[pl.ds(0, width * DIM)],
                              out_hbm.at[pl.ds(0, width * DIM)],
                              semo[u]).wait()

    iota33 = iota * (DIM + 1)

    def extract(src, ov, width):
        for l0 in range(0, width, 16):
            base = iota33 + l0 * (DIM + 1)
            for d in range(DIM):
                row = src[d, pl.ds(l0, 16)]
                plsc.store_scatter(ov33, [base + d], row)

        for l in range(width):
            s = l * (DIM + 1)
            t = l * DIM
            ov[pl.ds(t, 16)] = ov33[pl.ds(s, 16)]
            ov[pl.ds(t + 16, 16)] = ov33[pl.ds(s + 16, 16)]

    fire(0, 0)

    @pl.loop(0, BPW, step=2)
    def pair(b):
        fire(b + 1, 1)
        drain(0)

        @pl.when(b >= 2)
        def _():
            drain_out(0, TB)

        extract(in0, ov0, TB)
        fire_out(b, 0, TB)

        @pl.when(jnp.logical_or(b + 2 < BPW, extra))
        def _():
            fire(b + 2, 0)

        drain(1)

        @pl.when(b >= 2)
        def _():
            drain_out(1, TB)

        extract(in1, ov1, TB)
        fire_out(b + 1, 1, TB)

    @pl.when(extra)
    def _():
        drain(0)
        drain_out(0, TB)
        extract(in0, ov0, TB)
        fire_out(BPW, 0, TB)

    @pl.when(tailw)
    def _():
        pltpu.sync_copy(tt_hbm.at[pl.ds(0, DIM), pl.ds(NBF * TB, TAIL)],
                        int_)
        drain_out(1, TB)
        extract(int_, ov1, TAIL)
        pltpu.async_copy(ov1.at[pl.ds(0, TAIL * DIM)],
                         out_hbm.at[pl.ds(NBF * TB * DIM, TAIL * DIM)],
                         semo1)

    drain_out(0, TB)

    @pl.when(tailw)
    def _():
        drain_out(1, TAIL)

    @pl.when(jnp.logical_not(tailw))
    def _():
        drain_out(1, TB)


def _sc_transpose(table):
    mesh = plsc.VectorSubcoreMesh(
        core_axis_name="c", subcore_axis_name="s",
        num_cores=NC, num_subcores=NS)
    f = pl.kernel(
        _tr_kernel,
        out_type=jax.ShapeDtypeStruct((VOCAB * DIM,), jnp.float32),
        mesh=mesh,
        scratch_types=[
            pltpu.VMEM((DIM, TB), jnp.float32),
            pltpu.VMEM((DIM, TB), jnp.float32),
            pltpu.VMEM((DIM, TAIL), jnp.float32),
            pltpu.VMEM((TB * (DIM + 1),), jnp.float32),
            pltpu.VMEM((TB * DIM,), jnp.float32),
            pltpu.VMEM((TB * DIM,), jnp.float32),
            pltpu.SemaphoreType.DMA,
            pltpu.SemaphoreType.DMA,
            pltpu.SemaphoreType.DMA,
            pltpu.SemaphoreType.DMA,
        ],
        compiler_params=pltpu.CompilerParams(needs_layout_passes=False),
    )
    return f(table.T)


@jax.jit
def _run(keys, tflat):
    table = tflat.reshape(VOCAB, DIM)
    mesh = plsc.VectorSubcoreMesh(
        core_axis_name="c", subcore_axis_name="s",
        num_cores=NC, num_subcores=NS)
    f = pl.kernel(
        _emb_kernel,
        out_type=jax.ShapeDtypeStruct((SEGS, DIM), jnp.float32),
        mesh=mesh,
        scratch_types=[
            pltpu.VMEM((CKEY,), jnp.int32),
            pltpu.VMEM((CKEY,), jnp.int32),
            pltpu.VMEM((CKEY, DIM), jnp.float32),
            pltpu.VMEM((CKEY, DIM), jnp.float32),
            pltpu.VMEM((CSEG, DIM), jnp.float32),
            pltpu.VMEM((CSEG, DIM), jnp.float32),
            pltpu.SemaphoreType.DMA,
            pltpu.SemaphoreType.DMA,
        ],
        compiler_params=pltpu.CompilerParams(use_tc_tiling_on_sc=False),
    )
    return f(table, keys)


def kernel(inputs, table):
    keys = inputs.reshape(-1)
    out = _run(keys, _sc_transpose(table))
    return out.reshape(BATCH, SLOT, DIM)
